# Initial kernel scaffold; baseline (speedup 1.0000x reference)
import jax
import jax.numpy as jnp
from jax.experimental import pallas as pl


def kernel(x, edge_index, batch, edge_weight, Wrel0, brel0, Wroot0, Wrel1, brel1, Wroot1, Wrel2, brel2, Wroot2, Wl1, bl1, Wl2, bl2):
    raise NotImplementedError("write your pallas kernel here")


# speedup vs baseline: 1.0294x; 1.0294x over previous
import jax
import jax.numpy as jnp
from jax.experimental import pallas as pl


def _mlp_body(pooled_ref, wl1_ref, bl1_ref, wl2_ref, bl2_ref, out_ref):
    h = jnp.maximum(pooled_ref[...] @ wl1_ref[...].T + bl1_ref[...], 0.0)
    out_ref[...] = jnp.sum(h * wl2_ref[...], axis=1, keepdims=True) + bl2_ref[0, 0]


def kernel(x, edge_index, batch, edge_weight, Wrel0, brel0, Wroot0, Wrel1, brel1, Wroot1, Wrel2, brel2, Wroot2, Wl1, bl1, Wl2, bl2):
    src = edge_index[0]
    dst = edge_index[1]

    def conv(h, Wrel, brel, Wroot):
        xr = h @ Wrel.T
        msg = xr[src] * edge_weight[:, None]
        agg = jax.ops.segment_sum(msg, dst, num_segments=h.shape[0])
        return jnp.maximum(agg + brel + h @ Wroot.T, 0.0)

    h = conv(x, Wrel0, brel0, Wroot0)
    h = conv(h, Wrel1, brel1, Wroot1)
    h = conv(h, Wrel2, brel2, Wroot2)
    pooled = jax.ops.segment_max(h, batch, num_segments=64)
    out = pl.pallas_call(
        _mlp_body,
        out_shape=jax.ShapeDtypeStruct((64, 1), jnp.float32),
    )(pooled, Wl1, bl1.reshape(1, -1), Wl2, bl2.reshape(1, 1))
    return out


# speedup vs baseline: 6.5566x; 6.3693x over previous
import functools

import jax
import jax.numpy as jnp
from jax import lax
from jax.experimental import pallas as pl
from jax.experimental.pallas import tpu as pltpu
from jax.experimental.pallas import tpu_sc as plsc

_NC = 2
_NS = 16
_NW = _NC * _NS
_CS = 80


def _sc_scatter_body(xr_hbm, srcm_hbm, dstm_hbm, wm_hbm, out_hbm,
                     src_v, dst_v, w_v, rows_v, stage_v, acc_sh, sem):
    n_pad, h = acc_sh.shape
    nch = src_v.shape[0]
    rows_per_tile = n_pad // _NS
    cid = lax.axis_index("c")
    sid = lax.axis_index("s")
    wid = sid * _NC + cid

    pltpu.sync_copy(srcm_hbm.at[wid], src_v)
    pltpu.sync_copy(dstm_hbm.at[wid], dst_v)
    pltpu.sync_copy(wm_hbm.at[wid], w_v)

    def _zero_row(i, _):
        for c in range(h // 16):
            stage_v[i, pl.ds(c * 16, 16)] = jnp.zeros((16,), jnp.float32)
        return 0
    lax.fori_loop(0, rows_per_tile, _zero_row, 0)
    pltpu.sync_copy(stage_v, acc_sh.at[pl.ds(sid * rows_per_tile, rows_per_tile)])
    plsc.subcore_barrier()

    def _chunk(j, _):
        pltpu.async_copy(xr_hbm.at[src_v.at[j]], rows_v, sem).wait()
        for k in range(_CS // 16):
            wvec = w_v[j, pl.ds(k * 16, 16)]
            for l in range(16):
                e = k * 16 + l
                wb = jnp.full((16,), wvec[l], jnp.float32)
                for c in range(h // 16):
                    sl = pl.ds(c * 16, 16)
                    rows_v[e, sl] = rows_v[e, sl] * wb
        pltpu.sync_copy(rows_v, acc_sh.at[dst_v.at[j]], add=True)
        return 0
    lax.fori_loop(0, nch, _chunk, 0)

    plsc.subcore_barrier()
    rsl = pl.ds(sid * rows_per_tile, rows_per_tile)
    pltpu.sync_copy(acc_sh.at[rsl], stage_v)
    pltpu.sync_copy(stage_v, out_hbm.at[cid, rsl])


def _sc_scatter(xr, srcm, dstm, wm):
    n, h = xr.shape
    nch = srcm.shape[1]
    cs = srcm.shape[2]
    rpt = (-(-n // _NS) + 7) // 8 * 8
    n_pad = rpt * _NS
    mesh = plsc.VectorSubcoreMesh(core_axis_name="c", subcore_axis_name="s")
    f = pl.kernel(
        _sc_scatter_body, mesh=mesh,
        compiler_params=pltpu.CompilerParams(use_tc_tiling_on_sc=False),
        out_type=jax.ShapeDtypeStruct((_NC, n_pad, h), jnp.float32),
        scratch_types=[
            pltpu.VMEM((nch, cs), jnp.int32),
            pltpu.VMEM((nch, cs), jnp.int32),
            pltpu.VMEM((nch, cs), jnp.float32),
            pltpu.VMEM((cs, h), jnp.float32),
            pltpu.VMEM((rpt, h), jnp.float32),
            pltpu.VMEM_SHARED((n_pad, h), jnp.float32),
            pltpu.SemaphoreType.DMA,
        ],
    )
    return f(xr, srcm, dstm, wm)


def _mm2_body(x_ref, a_ref, b_ref, xr_ref, xo_ref):
    x = x_ref[...]
    xr_ref[...] = jnp.dot(x, a_ref[...], preferred_element_type=jnp.float32)
    xo_ref[...] = jnp.dot(x, b_ref[...], preferred_element_type=jnp.float32)


def _mm2(x, wrelT, wrootT, rows_blk):
    n, fin = x.shape
    hh = wrelT.shape[1]
    grid = n // rows_blk
    return pl.pallas_call(
        _mm2_body,
        grid=(grid,),
        in_specs=[
            pl.BlockSpec((rows_blk, fin), lambda i: (i, 0)),
            pl.BlockSpec((fin, hh), lambda i: (0, 0)),
            pl.BlockSpec((fin, hh), lambda i: (0, 0)),
        ],
        out_specs=[
            pl.BlockSpec((rows_blk, hh), lambda i: (i, 0)),
            pl.BlockSpec((rows_blk, hh), lambda i: (i, 0)),
        ],
        out_shape=[
            jax.ShapeDtypeStruct((n, hh), jnp.float32),
            jax.ShapeDtypeStruct((n, hh), jnp.float32),
        ],
    )(x, wrelT, wrootT)


def _combine_mm2_body(p_ref, xo_ref, b_ref, a2_ref, b2_ref, xr_ref, xo2_ref):
    hcur = jnp.maximum(p_ref[0] + p_ref[1] + xo_ref[...] + b_ref[...], 0.0)
    xr_ref[...] = jnp.dot(hcur, a2_ref[...], preferred_element_type=jnp.float32)
    xo2_ref[...] = jnp.dot(hcur, b2_ref[...], preferred_element_type=jnp.float32)


def _combine_mm2(p, xo, brel, wrelT, wrootT, rows_blk):
    n, hh = xo.shape
    grid = n // rows_blk
    return pl.pallas_call(
        _combine_mm2_body,
        grid=(grid,),
        in_specs=[
            pl.BlockSpec((_NC, rows_blk, hh), lambda i: (0, i, 0)),
            pl.BlockSpec((rows_blk, hh), lambda i: (i, 0)),
            pl.BlockSpec((1, hh), lambda i: (0, 0)),
            pl.BlockSpec((hh, hh), lambda i: (0, 0)),
            pl.BlockSpec((hh, hh), lambda i: (0, 0)),
        ],
        out_specs=[
            pl.BlockSpec((rows_blk, hh), lambda i: (i, 0)),
            pl.BlockSpec((rows_blk, hh), lambda i: (i, 0)),
        ],
        out_shape=[
            jax.ShapeDtypeStruct((n, hh), jnp.float32),
            jax.ShapeDtypeStruct((n, hh), jnp.float32),
        ],
    )(p, xo, brel, wrelT, wrootT)


def _combine_body(p_ref, xo_ref, b_ref, h_ref):
    h_ref[...] = jnp.maximum(p_ref[0] + p_ref[1] + xo_ref[...] + b_ref[...], 0.0)


def _combine(p, xo, brel, rows_blk):
    n, hh = xo.shape
    grid = n // rows_blk
    return pl.pallas_call(
        _combine_body,
        grid=(grid,),
        in_specs=[
            pl.BlockSpec((_NC, rows_blk, hh), lambda i: (0, i, 0)),
            pl.BlockSpec((rows_blk, hh), lambda i: (i, 0)),
            pl.BlockSpec((1, hh), lambda i: (0, 0)),
        ],
        out_specs=pl.BlockSpec((rows_blk, hh), lambda i: (i, 0)),
        out_shape=jax.ShapeDtypeStruct((n, hh), jnp.float32),
    )(p, xo, brel)


def _pool_body(h_ref, b_ref, out_ref):
    g = pl.program_id(0)
    masked = jnp.where(b_ref[...] == g, h_ref[...], -jnp.inf)
    out_ref[...] = jnp.max(masked, axis=0, keepdims=True)[None]


def _pool(h3, batchfull, num_graphs):
    n, hh = h3.shape
    out3 = pl.pallas_call(
        _pool_body,
        grid=(num_graphs,),
        in_specs=[
            pl.BlockSpec((n, hh), lambda g: (0, 0)),
            pl.BlockSpec((n, hh), lambda g: (0, 0)),
        ],
        out_specs=pl.BlockSpec((1, 1, hh), lambda g: (g, 0, 0)),
        out_shape=jax.ShapeDtypeStruct((num_graphs, 1, hh), jnp.float32),
    )(h3, batchfull)
    return out3.reshape(num_graphs, hh)


def _mlp_body(pooled_ref, wl1_ref, bl1_ref, wl2_ref, bl2_ref, out_ref):
    hid = jnp.maximum(
        jnp.dot(pooled_ref[...], wl1_ref[...], preferred_element_type=jnp.float32)
        + bl1_ref[...], 0.0)
    out_ref[...] = jnp.sum(hid * wl2_ref[...], axis=1, keepdims=True) + bl2_ref[0, 0]


def _mlp(pooled, wl1T, bl1, wl2, bl2):
    g = pooled.shape[0]
    return pl.pallas_call(
        _mlp_body,
        out_shape=jax.ShapeDtypeStruct((g, 1), jnp.float32),
    )(pooled, wl1T, bl1.reshape(1, -1), wl2, bl2.reshape(1, 1))


def kernel(x, edge_index, batch, edge_weight, Wrel0, brel0, Wroot0, Wrel1, brel1, Wroot1, Wrel2, brel2, Wroot2, Wl1, bl1, Wl2, bl2):
    n, fin = x.shape
    e = edge_weight.shape[0]
    hh = Wrel0.shape[0]
    num_graphs = 64
    ept = e // _NW
    nch = ept // _CS

    srcm = edge_index[0].reshape(_NW, nch, _CS)
    dstm = edge_index[1].reshape(_NW, nch, _CS)
    wm = edge_weight.reshape(_NW, nch, _CS)
    batchfull = jnp.broadcast_to(batch[:, None], (n, hh)).astype(jnp.int32)

    rows_blk = 2000

    xr, xo = _mm2(x, Wrel0.T, Wroot0.T, rows_blk)
    p = _sc_scatter(xr, srcm, dstm, wm)
    xr, xo = _combine_mm2(p, xo, brel0.reshape(1, -1), Wrel1.T, Wroot1.T, rows_blk)
    p = _sc_scatter(xr, srcm, dstm, wm)
    xr, xo = _combine_mm2(p, xo, brel1.reshape(1, -1), Wrel2.T, Wroot2.T, rows_blk)
    p = _sc_scatter(xr, srcm, dstm, wm)
    h3 = _combine(p, xo, brel2.reshape(1, -1), rows_blk)
    pooled = _pool(h3, batchfull, num_graphs)
    return _mlp(pooled, Wl1.T, bl1, Wl2, bl2)


# speedup vs baseline: 7.5457x; 1.1509x over previous
import functools

import jax
import jax.numpy as jnp
from jax import lax
from jax.experimental import pallas as pl
from jax.experimental.pallas import tpu as pltpu
from jax.experimental.pallas import tpu_sc as plsc

_NC = 2
_NS = 16
_NW = _NC * _NS
_CS = 80
_GB = 5


def _sc_scatter_body(xr_hbm, srcm_hbm, dstm_hbm, wm_hbm, out_hbm,
                     src_v, dst_v, w_v, rows_v, acc, gsem, ssem):
    n_pad, h = acc.shape
    rows_per_tile = n_pad // _NS
    nch = src_v.shape[0]
    cid = lax.axis_index("c")
    sid = lax.axis_index("s")
    wid = sid * _NC + cid

    pltpu.sync_copy(srcm_hbm.at[wid], src_v)
    pltpu.sync_copy(dstm_hbm.at[wid], dst_v)
    pltpu.sync_copy(wm_hbm.at[wid], w_v)

    def _zero_row(i, _):
        for c in range(h // 16):
            rows_v[0, i, pl.ds(c * 16, 16)] = jnp.zeros((16,), jnp.float32)
        return 0
    lax.fori_loop(0, _CS, _zero_row, 0)
    for t in range(rows_per_tile // _CS):
        pltpu.sync_copy(rows_v.at[0],
                        acc.at[pl.ds(sid * rows_per_tile + t * _CS, _CS)])
    plsc.subcore_barrier()

    def _group(g, _):
        j0 = g * _GB
        gh = [pltpu.async_copy(xr_hbm.at[src_v.at[j0 + b]], rows_v.at[b], gsem)
              for b in range(_GB)]
        for hdl in gh:
            hdl.wait()
        for b in range(_GB):
            for k in range(_CS // 16):
                wvec = w_v[j0 + b, pl.ds(k * 16, 16)]
                for l in range(16):
                    e = k * 16 + l
                    wb = jnp.full((16,), wvec[l], jnp.float32)
                    for c in range(h // 16):
                        sl = pl.ds(c * 16, 16)
                        rows_v[b, e, sl] = rows_v[b, e, sl] * wb
        sh = [pltpu.async_copy(rows_v.at[b], acc.at[dst_v.at[j0 + b]], ssem,
                               add=True) for b in range(_GB)]
        for hdl in sh:
            hdl.wait()
        return 0
    lax.fori_loop(0, nch // _GB, _group, 0)
    plsc.subcore_barrier()
    rsl = pl.ds(sid * rows_per_tile, rows_per_tile)
    pltpu.sync_copy(acc.at[rsl], out_hbm.at[cid, rsl])


def _sc_scatter(xr, srcm, dstm, wm):
    n, h = xr.shape
    nch = srcm.shape[1]
    cs = srcm.shape[2]
    rpt = -(-(-(-n // _NS)) // cs) * cs
    n_pad = rpt * _NS
    mesh = plsc.VectorSubcoreMesh(core_axis_name="c", subcore_axis_name="s")
    f = pl.kernel(
        _sc_scatter_body, mesh=mesh,
        compiler_params=pltpu.CompilerParams(use_tc_tiling_on_sc=False),
        out_type=jax.ShapeDtypeStruct((_NC, n_pad, h), jnp.float32),
        scratch_types=[
            pltpu.VMEM((nch, cs), jnp.int32),
            pltpu.VMEM((nch, cs), jnp.int32),
            pltpu.VMEM((nch, cs), jnp.float32),
            pltpu.VMEM((_GB, cs, h), jnp.float32),
            pltpu.VMEM_SHARED((n_pad, h), jnp.float32),
            pltpu.SemaphoreType.DMA,
            pltpu.SemaphoreType.DMA,
        ],
    )
    return f(xr, srcm, dstm, wm)


def _mm2_body(x_ref, a_ref, b_ref, xr_ref, xo_ref):
    x = x_ref[...]
    xr_ref[...] = jnp.dot(x, a_ref[...], preferred_element_type=jnp.float32)
    xo_ref[...] = jnp.dot(x, b_ref[...], preferred_element_type=jnp.float32)


def _mm2(x, wrelT, wrootT, rows_blk):
    n, fin = x.shape
    hh = wrelT.shape[1]
    grid = n // rows_blk
    return pl.pallas_call(
        _mm2_body,
        grid=(grid,),
        in_specs=[
            pl.BlockSpec((rows_blk, fin), lambda i: (i, 0)),
            pl.BlockSpec((fin, hh), lambda i: (0, 0)),
            pl.BlockSpec((fin, hh), lambda i: (0, 0)),
        ],
        out_specs=[
            pl.BlockSpec((rows_blk, hh), lambda i: (i, 0)),
            pl.BlockSpec((rows_blk, hh), lambda i: (i, 0)),
        ],
        out_shape=[
            jax.ShapeDtypeStruct((n, hh), jnp.float32),
            jax.ShapeDtypeStruct((n, hh), jnp.float32),
        ],
    )(x, wrelT, wrootT)


def _combine_mm2_body(p_ref, xo_ref, b_ref, a2_ref, b2_ref, xr_ref, xo2_ref):
    hcur = jnp.maximum(p_ref[0] + p_ref[1] + xo_ref[...] + b_ref[...], 0.0)
    xr_ref[...] = jnp.dot(hcur, a2_ref[...], preferred_element_type=jnp.float32)
    xo2_ref[...] = jnp.dot(hcur, b2_ref[...], preferred_element_type=jnp.float32)


def _combine_mm2(p, xo, brel, wrelT, wrootT, rows_blk):
    n, hh = xo.shape
    grid = n // rows_blk
    return pl.pallas_call(
        _combine_mm2_body,
        grid=(grid,),
        in_specs=[
            pl.BlockSpec((_NC, rows_blk, hh), lambda i: (0, i, 0)),
            pl.BlockSpec((rows_blk, hh), lambda i: (i, 0)),
            pl.BlockSpec((1, hh), lambda i: (0, 0)),
            pl.BlockSpec((hh, hh), lambda i: (0, 0)),
            pl.BlockSpec((hh, hh), lambda i: (0, 0)),
        ],
        out_specs=[
            pl.BlockSpec((rows_blk, hh), lambda i: (i, 0)),
            pl.BlockSpec((rows_blk, hh), lambda i: (i, 0)),
        ],
        out_shape=[
            jax.ShapeDtypeStruct((n, hh), jnp.float32),
            jax.ShapeDtypeStruct((n, hh), jnp.float32),
        ],
    )(p, xo, brel, wrelT, wrootT)


def _combine_body(p_ref, xo_ref, b_ref, h_ref):
    h_ref[...] = jnp.maximum(p_ref[0] + p_ref[1] + xo_ref[...] + b_ref[...], 0.0)


def _combine(p, xo, brel, rows_blk):
    n, hh = xo.shape
    grid = n // rows_blk
    return pl.pallas_call(
        _combine_body,
        grid=(grid,),
        in_specs=[
            pl.BlockSpec((_NC, rows_blk, hh), lambda i: (0, i, 0)),
            pl.BlockSpec((rows_blk, hh), lambda i: (i, 0)),
            pl.BlockSpec((1, hh), lambda i: (0, 0)),
        ],
        out_specs=pl.BlockSpec((rows_blk, hh), lambda i: (i, 0)),
        out_shape=jax.ShapeDtypeStruct((n, hh), jnp.float32),
    )(p, xo, brel)


def _pool_body(h_ref, b_ref, out_ref):
    g = pl.program_id(0)
    masked = jnp.where(b_ref[...] == g, h_ref[...], -jnp.inf)
    out_ref[...] = jnp.max(masked, axis=0, keepdims=True)[None]


def _pool(h3, batchfull, num_graphs):
    n, hh = h3.shape
    out3 = pl.pallas_call(
        _pool_body,
        grid=(num_graphs,),
        in_specs=[
            pl.BlockSpec((n, hh), lambda g: (0, 0)),
            pl.BlockSpec((n, hh), lambda g: (0, 0)),
        ],
        out_specs=pl.BlockSpec((1, 1, hh), lambda g: (g, 0, 0)),
        out_shape=jax.ShapeDtypeStruct((num_graphs, 1, hh), jnp.float32),
    )(h3, batchfull)
    return out3.reshape(num_graphs, hh)


def _mlp_body(pooled_ref, wl1_ref, bl1_ref, wl2_ref, bl2_ref, out_ref):
    hid = jnp.maximum(
        jnp.dot(pooled_ref[...], wl1_ref[...], preferred_element_type=jnp.float32)
        + bl1_ref[...], 0.0)
    out_ref[...] = jnp.sum(hid * wl2_ref[...], axis=1, keepdims=True) + bl2_ref[0, 0]


def _mlp(pooled, wl1T, bl1, wl2, bl2):
    g = pooled.shape[0]
    return pl.pallas_call(
        _mlp_body,
        out_shape=jax.ShapeDtypeStruct((g, 1), jnp.float32),
    )(pooled, wl1T, bl1.reshape(1, -1), wl2, bl2.reshape(1, 1))


def kernel(x, edge_index, batch, edge_weight, Wrel0, brel0, Wroot0, Wrel1, brel1, Wroot1, Wrel2, brel2, Wroot2, Wl1, bl1, Wl2, bl2):
    n, fin = x.shape
    e = edge_weight.shape[0]
    hh = Wrel0.shape[0]
    num_graphs = 64
    ept = e // _NW
    nch = ept // _CS

    srcm = edge_index[0].reshape(_NW, nch, _CS)
    dstm = edge_index[1].reshape(_NW, nch, _CS)
    wm = edge_weight.reshape(_NW, nch, _CS)
    batchfull = jnp.broadcast_to(batch[:, None], (n, hh)).astype(jnp.int32)

    rows_blk = 2000

    xr, xo = _mm2(x, Wrel0.T, Wroot0.T, rows_blk)
    p = _sc_scatter(xr, srcm, dstm, wm)
    xr, xo = _combine_mm2(p, xo, brel0.reshape(1, -1), Wrel1.T, Wroot1.T, rows_blk)
    p = _sc_scatter(xr, srcm, dstm, wm)
    xr, xo = _combine_mm2(p, xo, brel1.reshape(1, -1), Wrel2.T, Wroot2.T, rows_blk)
    p = _sc_scatter(xr, srcm, dstm, wm)
    h3 = _combine(p, xo, brel2.reshape(1, -1), rows_blk)
    pooled = _pool(h3, batchfull, num_graphs)
    return _mlp(pooled, Wl1.T, bl1, Wl2, bl2)


# speedup vs baseline: 8.7671x; 1.1619x over previous
import functools

import jax
import jax.numpy as jnp
from jax import lax
from jax.experimental import pallas as pl
from jax.experimental.pallas import tpu as pltpu
from jax.experimental.pallas import tpu_sc as plsc

_NC = 2
_NS = 16
_NW = _NC * _NS
_CS = 80
_GB = 5
_NIDLE = 7


def _sc_scatter_body(xr_hbm, srcm_hbm, dstm_hbm, wm_hbm, out_hbm,
                     src_v, dst_v, w_v, rows_v, acc, gsem, ssem):
    n_pad, h = acc.shape
    rows_per_tile = n_pad // _NS
    nch = src_v.shape[0]
    cid = lax.axis_index("c")
    sid = lax.axis_index("s")
    wid = sid * _NC + cid

    pltpu.sync_copy(srcm_hbm.at[wid], src_v)
    pltpu.sync_copy(dstm_hbm.at[wid], dst_v)
    pltpu.sync_copy(wm_hbm.at[wid], w_v)

    def _zero_row(i, _):
        for c in range(h // 16):
            rows_v[0, i, pl.ds(c * 16, 16)] = jnp.zeros((16,), jnp.float32)
        return 0
    lax.fori_loop(0, _CS, _zero_row, 0)
    for t in range(rows_per_tile // _CS):
        pltpu.sync_copy(rows_v.at[0],
                        acc.at[pl.ds(sid * rows_per_tile + t * _CS, _CS)])
    plsc.subcore_barrier()

    def _group(g, _):
        j0 = g * _GB
        gh = [pltpu.async_copy(xr_hbm.at[src_v.at[j0 + b]], rows_v.at[b], gsem)
              for b in range(_GB)]
        for hdl in gh:
            hdl.wait()
        for b in range(_GB):
            for k in range(_CS // 16):
                wvec = w_v[j0 + b, pl.ds(k * 16, 16)]
                for l in range(16):
                    e = k * 16 + l
                    wb = jnp.full((16,), wvec[l], jnp.float32)
                    for c in range(h // 16):
                        sl = pl.ds(c * 16, 16)
                        rows_v[b, e, sl] = rows_v[b, e, sl] * wb
        sh = [pltpu.async_copy(rows_v.at[b], acc.at[dst_v.at[j0 + b]], ssem,
                               add=True) for b in range(_GB)]
        for hdl in sh:
            hdl.wait()
        return 0
    lax.fori_loop(0, nch // _GB, _group, 0)
    plsc.subcore_barrier()
    rsl = pl.ds(sid * rows_per_tile, rows_per_tile)
    pltpu.sync_copy(acc.at[rsl], out_hbm.at[cid, rsl])


def _sc_scatter(xr, srcm, dstm, wm):
    n, h = xr.shape
    nch = srcm.shape[1]
    cs = srcm.shape[2]
    rpt = -(-(-(-n // _NS)) // cs) * cs
    n_pad = rpt * _NS
    mesh = plsc.VectorSubcoreMesh(core_axis_name="c", subcore_axis_name="s")
    f = pl.kernel(
        _sc_scatter_body, mesh=mesh,
        compiler_params=pltpu.CompilerParams(use_tc_tiling_on_sc=False),
        out_type=jax.ShapeDtypeStruct((_NC, n_pad, h), jnp.float32),
        scratch_types=[
            pltpu.VMEM((nch, cs), jnp.int32),
            pltpu.VMEM((nch, cs), jnp.int32),
            pltpu.VMEM((nch, cs), jnp.float32),
            pltpu.VMEM((_GB, cs, h), jnp.float32),
            pltpu.VMEM_SHARED((n_pad, h), jnp.float32),
            pltpu.SemaphoreType.DMA,
            pltpu.SemaphoreType.DMA,
        ],
    )
    return f(xr, srcm, dstm, wm)


def _mm2_body(x_ref, a_ref, b_ref, xr_ref, xo_ref):
    x = x_ref[...]
    xr_ref[...] = jnp.dot(x, a_ref[...], preferred_element_type=jnp.float32)
    xo_ref[...] = jnp.dot(x, b_ref[...], preferred_element_type=jnp.float32)


def _mm2(x, wrelT, wrootT, rows_blk):
    n, fin = x.shape
    hh = wrelT.shape[1]
    grid = n // rows_blk
    return pl.pallas_call(
        _mm2_body,
        grid=(grid,),
        in_specs=[
            pl.BlockSpec((rows_blk, fin), lambda i: (i, 0)),
            pl.BlockSpec((fin, hh), lambda i: (0, 0)),
            pl.BlockSpec((fin, hh), lambda i: (0, 0)),
        ],
        out_specs=[
            pl.BlockSpec((rows_blk, hh), lambda i: (i, 0)),
            pl.BlockSpec((rows_blk, hh), lambda i: (i, 0)),
        ],
        out_shape=[
            jax.ShapeDtypeStruct((n, hh), jnp.float32),
            jax.ShapeDtypeStruct((n, hh), jnp.float32),
        ],
    )(x, wrelT, wrootT)


def _combine_mm2_body(p_ref, xo_ref, b_ref, a2_ref, b2_ref, xr_ref, xo2_ref):
    hcur = jnp.maximum(p_ref[0] + p_ref[1] + xo_ref[...] + b_ref[...], 0.0)
    xr_ref[...] = jnp.dot(hcur, a2_ref[...], preferred_element_type=jnp.float32)
    xo2_ref[...] = jnp.dot(hcur, b2_ref[...], preferred_element_type=jnp.float32)


def _combine_mm2(p, xo, brel, wrelT, wrootT, rows_blk):
    n, hh = xo.shape
    grid = n // rows_blk
    return pl.pallas_call(
        _combine_mm2_body,
        grid=(grid,),
        in_specs=[
            pl.BlockSpec((_NC, rows_blk, hh), lambda i: (0, i, 0)),
            pl.BlockSpec((rows_blk, hh), lambda i: (i, 0)),
            pl.BlockSpec((1, hh), lambda i: (0, 0)),
            pl.BlockSpec((hh, hh), lambda i: (0, 0)),
            pl.BlockSpec((hh, hh), lambda i: (0, 0)),
        ],
        out_specs=[
            pl.BlockSpec((rows_blk, hh), lambda i: (i, 0)),
            pl.BlockSpec((rows_blk, hh), lambda i: (i, 0)),
        ],
        out_shape=[
            jax.ShapeDtypeStruct((n, hh), jnp.float32),
            jax.ShapeDtypeStruct((n, hh), jnp.float32),
        ],
    )(p, xo, brel, wrelT, wrootT)


def _sc_pool_body(p_hbm, xo_hbm, brel_hbm, batch_hbm, out_hbm,
                  p0_v, p1_v, xo_v, b_v, bid_v, part_v, sem0, sem1, sem2, sem3):
    rows_pw, h = p0_v.shape
    ng = part_v.shape[0]
    cid = lax.axis_index("c")
    sid = lax.axis_index("s")
    wid = sid * _NC + cid
    nact = _NW - _NIDLE
    ninf = jnp.full((16,), -jnp.inf, jnp.float32)

    for i in range(ng):
        for c in range(h // 16):
            part_v[i, pl.ds(c * 16, 16)] = ninf

    @pl.when(wid < nact)
    def _active():
        base = wid * rows_pw
        rs = pl.ds(base, rows_pw)
        cp = [pltpu.async_copy(p_hbm.at[0, rs], p0_v, sem0),
              pltpu.async_copy(p_hbm.at[1, rs], p1_v, sem1),
              pltpu.async_copy(xo_hbm.at[rs], xo_v, sem2),
              pltpu.async_copy(batch_hbm.at[rs], bid_v, sem3)]
        pltpu.sync_copy(brel_hbm, b_v)
        for hdl in cp:
            hdl.wait()

        ids0 = bid_v[pl.ds(0, 16)]

        def _row16(k, carry):
            ids16 = bid_v[pl.ds(k * 16, 16)]
            for l in range(16):
                prev = carry[0]
                m = carry[1:]
                gid = ids16[l]
                newseg = gid != prev
                nsv = jnp.full((16,), newseg)
                for c in range(h // 16):
                    cols = c * 16 + lax.iota(jnp.int32, 16)
                    rows = jnp.full((16,), prev, jnp.int32)
                    plsc.store_scatter(part_v, [rows, cols], m[c])


                i = k * 16 + l
                mn = []
                for c in range(h // 16):
                    sl = pl.ds(c * 16, 16)
                    hrow = jnp.maximum(
                        p0_v[i, sl] + p1_v[i, sl] + xo_v[i, sl] + b_v[0, sl],
                        0.0)
                    mc = jnp.where(nsv, ninf, m[c])
                    mn.append(jnp.maximum(mc, hrow))
                carry = (gid,) + tuple(mn)
            return carry

        carry = lax.fori_loop(0, rows_pw // 16, _row16,
                              (ids0[0],) + tuple(ninf for _ in range(h // 16)))
        prev = carry[0]
        for c in range(h // 16):
            cols = c * 16 + lax.iota(jnp.int32, 16)
            rows = jnp.full((16,), prev, jnp.int32)
            plsc.store_scatter(part_v, [rows, cols], carry[1 + c])

    pltpu.sync_copy(part_v, out_hbm.at[wid])


def _sc_pool(p, xo, brel, batch, num_graphs):
    n, hh = xo.shape
    nact = _NW - _NIDLE
    rows_pw = n // nact
    mesh = plsc.VectorSubcoreMesh(core_axis_name="c", subcore_axis_name="s")
    f = pl.kernel(
        _sc_pool_body, mesh=mesh,
        compiler_params=pltpu.CompilerParams(use_tc_tiling_on_sc=False,
                                             needs_layout_passes=False),
        out_type=jax.ShapeDtypeStruct((_NW, num_graphs, hh), jnp.float32),
        scratch_types=[
            pltpu.VMEM((rows_pw, hh), jnp.float32),
            pltpu.VMEM((rows_pw, hh), jnp.float32),
            pltpu.VMEM((rows_pw, hh), jnp.float32),
            pltpu.VMEM((1, hh), jnp.float32),
            pltpu.VMEM((rows_pw,), jnp.int32),
            pltpu.VMEM((num_graphs, hh), jnp.float32),
            pltpu.SemaphoreType.DMA,
            pltpu.SemaphoreType.DMA,
            pltpu.SemaphoreType.DMA,
            pltpu.SemaphoreType.DMA,
        ],
    )
    return f(p, xo, brel.reshape(1, -1), batch)


def _mlp_body(parts_ref, wl1_ref, bl1_ref, wl2_ref, bl2_ref, out_ref):
    pooled = parts_ref[0]
    for i in range(1, parts_ref.shape[0]):
        pooled = jnp.maximum(pooled, parts_ref[i])
    hid = jnp.maximum(
        jnp.dot(pooled, wl1_ref[...], preferred_element_type=jnp.float32)
        + bl1_ref[...], 0.0)
    out_ref[...] = jnp.sum(hid * wl2_ref[...], axis=1, keepdims=True) + bl2_ref[0, 0]


def _mlp(parts, wl1T, bl1, wl2, bl2):
    g = parts.shape[1]
    return pl.pallas_call(
        _mlp_body,
        out_shape=jax.ShapeDtypeStruct((g, 1), jnp.float32),
    )(parts, wl1T, bl1.reshape(1, -1), wl2, bl2.reshape(1, 1))


def kernel(x, edge_index, batch, edge_weight, Wrel0, brel0, Wroot0, Wrel1, brel1, Wroot1, Wrel2, brel2, Wroot2, Wl1, bl1, Wl2, bl2):
    n, fin = x.shape
    e = edge_weight.shape[0]
    hh = Wrel0.shape[0]
    num_graphs = 64
    ept = e // _NW
    nch = ept // _CS

    srcm = edge_index[0].reshape(_NW, nch, _CS)
    dstm = edge_index[1].reshape(_NW, nch, _CS)
    wm = edge_weight.reshape(_NW, nch, _CS)

    rows_blk = 2000

    xr, xo = _mm2(x, Wrel0.T, Wroot0.T, rows_blk)
    p = _sc_scatter(xr, srcm, dstm, wm)
    xr, xo = _combine_mm2(p, xo, brel0.reshape(1, -1), Wrel1.T, Wroot1.T, rows_blk)
    p = _sc_scatter(xr, srcm, dstm, wm)
    xr, xo = _combine_mm2(p, xo, brel1.reshape(1, -1), Wrel2.T, Wroot2.T, rows_blk)
    p = _sc_scatter(xr, srcm, dstm, wm)
    parts = _sc_pool(p, xo, brel2, batch, num_graphs)
    return _mlp(parts, Wl1.T, bl1, Wl2, bl2)


# speedup vs baseline: 14.8542x; 1.6943x over previous
import functools

import jax
import jax.numpy as jnp
from jax import lax
from jax.experimental import pallas as pl
from jax.experimental.pallas import tpu as pltpu
from jax.experimental.pallas import tpu_sc as plsc

_NC = 2
_NS = 16
_NW = _NC * _NS
_CS = 80
_GB = 5
_NIDLE = 7


def _sc_scatter_body(xr_hbm, srcm_hbm, dstm_hbm, wm_hbm, out_hbm,
                     srcr, dstr, dsts, wr, rows_v, acc, gsem, ssem, isem):
    n_pad, h = acc.shape
    rows_per_tile = n_pad // _NS
    ngr = srcm_hbm.shape[1]
    cid = lax.axis_index("c")
    sid = lax.axis_index("s")
    wid = sid * _NC + cid

    def idx_issue(g, s):
        pltpu.async_copy(srcm_hbm.at[wid, g], srcr.at[s], isem.at[s])
        pltpu.async_copy(dstm_hbm.at[wid, g], dstr.at[s], isem.at[s])
        pltpu.async_copy(wm_hbm.at[wid, g], wr.at[s], isem.at[s])

    def idx_drain(s):
        pltpu.make_async_copy(srcm_hbm.at[wid, 0], srcr.at[s], isem.at[s]).wait()
        pltpu.make_async_copy(dstm_hbm.at[wid, 0], dstr.at[s], isem.at[s]).wait()
        pltpu.make_async_copy(wm_hbm.at[wid, 0], wr.at[s], isem.at[s]).wait()

    def gat_issue(r):
        for b in range(_GB):
            pltpu.async_copy(xr_hbm.at[srcr.at[r, b]], rows_v.at[r, b],
                             gsem.at[r])

    def gat_drain(r):
        for b in range(_GB):
            pltpu.make_async_copy(xr_hbm.at[srcr.at[r, 0]], rows_v.at[r, 0],
                                  gsem.at[r]).wait()

    def sca_issue(r):
        for b in range(_GB):
            for k in range(_CS // 16):
                sl = pl.ds(k * 16, 16)
                dsts[r, b, sl] = dstr[r, b, sl]
        for b in range(_GB):
            pltpu.async_copy(rows_v.at[r, b], acc.at[dsts.at[r, b]],
                             ssem.at[r], add=True)

    def sca_drain(r):
        for b in range(_GB):
            pltpu.make_async_copy(rows_v.at[r, 0], acc.at[dsts.at[r, 0]],
                                  ssem.at[r]).wait()

    def scale(r):
        def _sb(b, _):
            for k in range(_CS // 16):
                wvec = wr[r, b, pl.ds(k * 16, 16)]
                for l in range(16):
                    e = k * 16 + l
                    wb = jnp.full((16,), wvec[l], jnp.float32)
                    for c in range(h // 16):
                        sl = pl.ds(c * 16, 16)
                        rows_v[r, b, e, sl] = rows_v[r, b, e, sl] * wb
            return 0
        lax.fori_loop(0, _GB, _sb, 0)

    def proc(g, r, drain_sca=True, fetch_gat=True, fetch_idx=True):
        rn = (r + 1) % 3
        rf = (r + 2) % 3
        if drain_sca:
            sca_drain(rn)
        if fetch_gat:
            idx_drain(rn)
            gat_issue(rn)
        if fetch_idx:
            idx_issue(g + 2, rf)
        gat_drain(r)
        scale(r)
        sca_issue(r)

    idx_issue(0, 0)
    idx_drain(0)
    gat_issue(0)
    idx_issue(1, 1)

    def _zero_row(i, _):
        for c in range(h // 16):
            rows_v[2, 0, i, pl.ds(c * 16, 16)] = jnp.zeros((16,), jnp.float32)
        return 0
    lax.fori_loop(0, _CS, _zero_row, 0)
    for t in range(rows_per_tile // _CS):
        pltpu.sync_copy(rows_v.at[2, 0],
                        acc.at[pl.ds(sid * rows_per_tile + t * _CS, _CS)])
    plsc.subcore_barrier()

    proc(0, 0, drain_sca=False)
    proc(1, 1, drain_sca=False)

    def _steady(i, _):
        g = 3 * i + 2
        proc(g, 2)
        proc(g + 1, 0)
        proc(g + 2, 1)
        return 0
    lax.fori_loop(0, (ngr - 4) // 3, _steady, 0)

    proc(ngr - 2, (ngr - 2) % 3, fetch_idx=False)
    proc(ngr - 1, (ngr - 1) % 3, fetch_gat=False, fetch_idx=False)
    sca_drain((ngr - 2) % 3)
    sca_drain((ngr - 1) % 3)

    plsc.subcore_barrier()
    rsl = pl.ds(sid * rows_per_tile, rows_per_tile)
    pltpu.sync_copy(acc.at[rsl], out_hbm.at[cid, rsl])


def _sc_scatter(xr, srcm, dstm, wm):
    n, h = xr.shape
    ngr = srcm.shape[1]
    cs = srcm.shape[3]
    rpt = -(-(-(-n // _NS)) // cs) * cs
    n_pad = rpt * _NS
    mesh = plsc.VectorSubcoreMesh(core_axis_name="c", subcore_axis_name="s")
    f = pl.kernel(
        _sc_scatter_body, mesh=mesh,
        compiler_params=pltpu.CompilerParams(use_tc_tiling_on_sc=False),
        out_type=jax.ShapeDtypeStruct((_NC, n_pad, h), jnp.float32),
        scratch_types=[
            pltpu.VMEM((3, _GB, cs), jnp.int32),
            pltpu.VMEM((3, _GB, cs), jnp.int32),
            pltpu.VMEM((3, _GB, cs), jnp.int32),
            pltpu.VMEM((3, _GB, cs), jnp.float32),
            pltpu.VMEM((3, _GB, cs, h), jnp.float32),
            pltpu.VMEM_SHARED((n_pad, h), jnp.float32),
            pltpu.SemaphoreType.DMA((3,)),
            pltpu.SemaphoreType.DMA((3,)),
            pltpu.SemaphoreType.DMA((3,)),
        ],
    )
    return f(xr, srcm, dstm, wm)


def _mm2_body(x_ref, a_ref, b_ref, xr_ref, xo_ref):
    x = x_ref[...]
    xr_ref[...] = jnp.dot(x, a_ref[...], preferred_element_type=jnp.float32)
    xo_ref[...] = jnp.dot(x, b_ref[...], preferred_element_type=jnp.float32)


def _mm2(x, wrelT, wrootT, rows_blk):
    n, fin = x.shape
    hh = wrelT.shape[1]
    grid = n // rows_blk
    return pl.pallas_call(
        _mm2_body,
        grid=(grid,),
        in_specs=[
            pl.BlockSpec((rows_blk, fin), lambda i: (i, 0)),
            pl.BlockSpec((fin, hh), lambda i: (0, 0)),
            pl.BlockSpec((fin, hh), lambda i: (0, 0)),
        ],
        out_specs=[
            pl.BlockSpec((rows_blk, hh), lambda i: (i, 0)),
            pl.BlockSpec((rows_blk, hh), lambda i: (i, 0)),
        ],
        out_shape=[
            jax.ShapeDtypeStruct((n, hh), jnp.float32),
            jax.ShapeDtypeStruct((n, hh), jnp.float32),
        ],
    )(x, wrelT, wrootT)


def _combine_mm2_body(p_ref, xo_ref, b_ref, a2_ref, b2_ref, xr_ref, xo2_ref):
    hcur = jnp.maximum(p_ref[0] + p_ref[1] + xo_ref[...] + b_ref[...], 0.0)
    xr_ref[...] = jnp.dot(hcur, a2_ref[...], preferred_element_type=jnp.float32)
    xo2_ref[...] = jnp.dot(hcur, b2_ref[...], preferred_element_type=jnp.float32)


def _combine_mm2(p, xo, brel, wrelT, wrootT, rows_blk):
    n, hh = xo.shape
    grid = n // rows_blk
    return pl.pallas_call(
        _combine_mm2_body,
        grid=(grid,),
        in_specs=[
            pl.BlockSpec((_NC, rows_blk, hh), lambda i: (0, i, 0)),
            pl.BlockSpec((rows_blk, hh), lambda i: (i, 0)),
            pl.BlockSpec((1, hh), lambda i: (0, 0)),
            pl.BlockSpec((hh, hh), lambda i: (0, 0)),
            pl.BlockSpec((hh, hh), lambda i: (0, 0)),
        ],
        out_specs=[
            pl.BlockSpec((rows_blk, hh), lambda i: (i, 0)),
            pl.BlockSpec((rows_blk, hh), lambda i: (i, 0)),
        ],
        out_shape=[
            jax.ShapeDtypeStruct((n, hh), jnp.float32),
            jax.ShapeDtypeStruct((n, hh), jnp.float32),
        ],
    )(p, xo, brel, wrelT, wrootT)


def _sc_pool_body(p_hbm, xo_hbm, brel_hbm, batch_hbm, out_hbm,
                  p0_v, p1_v, xo_v, b_v, bid_v, part_v, sem0, sem1, sem2, sem3):
    rows_pw, h = p0_v.shape
    ng = part_v.shape[0]
    cid = lax.axis_index("c")
    sid = lax.axis_index("s")
    wid = sid * _NC + cid
    nact = _NW - _NIDLE
    ninf = jnp.full((16,), -jnp.inf, jnp.float32)

    for i in range(ng):
        for c in range(h // 16):
            part_v[i, pl.ds(c * 16, 16)] = ninf

    @pl.when(wid < nact)
    def _active():
        base = wid * rows_pw
        rs = pl.ds(base, rows_pw)
        cp = [pltpu.async_copy(p_hbm.at[0, rs], p0_v, sem0),
              pltpu.async_copy(p_hbm.at[1, rs], p1_v, sem1),
              pltpu.async_copy(xo_hbm.at[rs], xo_v, sem2),
              pltpu.async_copy(batch_hbm.at[rs], bid_v, sem3)]
        pltpu.sync_copy(brel_hbm, b_v)
        for hdl in cp:
            hdl.wait()

        ids0 = bid_v[pl.ds(0, 16)]

        def _row16(k, carry):
            ids16 = bid_v[pl.ds(k * 16, 16)]
            for l in range(16):
                prev = carry[0]
                m = carry[1:]
                gid = ids16[l]
                newseg = gid != prev
                nsv = jnp.full((16,), newseg)
                for c in range(h // 16):
                    cols = c * 16 + lax.iota(jnp.int32, 16)
                    rows = jnp.full((16,), prev, jnp.int32)
                    plsc.store_scatter(part_v, [rows, cols], m[c])


                i = k * 16 + l
                mn = []
                for c in range(h // 16):
                    sl = pl.ds(c * 16, 16)
                    hrow = jnp.maximum(
                        p0_v[i, sl] + p1_v[i, sl] + xo_v[i, sl] + b_v[0, sl],
                        0.0)
                    mc = jnp.where(nsv, ninf, m[c])
                    mn.append(jnp.maximum(mc, hrow))
                carry = (gid,) + tuple(mn)
            return carry

        carry = lax.fori_loop(0, rows_pw // 16, _row16,
                              (ids0[0],) + tuple(ninf for _ in range(h // 16)))
        prev = carry[0]
        for c in range(h // 16):
            cols = c * 16 + lax.iota(jnp.int32, 16)
            rows = jnp.full((16,), prev, jnp.int32)
            plsc.store_scatter(part_v, [rows, cols], carry[1 + c])

    pltpu.sync_copy(part_v, out_hbm.at[wid])


def _sc_pool(p, xo, brel, batch, num_graphs):
    n, hh = xo.shape
    nact = _NW - _NIDLE
    rows_pw = n // nact
    mesh = plsc.VectorSubcoreMesh(core_axis_name="c", subcore_axis_name="s")
    f = pl.kernel(
        _sc_pool_body, mesh=mesh,
        compiler_params=pltpu.CompilerParams(use_tc_tiling_on_sc=False,
                                             needs_layout_passes=False),
        out_type=jax.ShapeDtypeStruct((_NW, num_graphs, hh), jnp.float32),
        scratch_types=[
            pltpu.VMEM((rows_pw, hh), jnp.float32),
            pltpu.VMEM((rows_pw, hh), jnp.float32),
            pltpu.VMEM((rows_pw, hh), jnp.float32),
            pltpu.VMEM((1, hh), jnp.float32),
            pltpu.VMEM((rows_pw,), jnp.int32),
            pltpu.VMEM((num_graphs, hh), jnp.float32),
            pltpu.SemaphoreType.DMA,
            pltpu.SemaphoreType.DMA,
            pltpu.SemaphoreType.DMA,
            pltpu.SemaphoreType.DMA,
        ],
    )
    return f(p, xo, brel.reshape(1, -1), batch)


def _mlp_body(parts_ref, wl1_ref, bl1_ref, wl2_ref, bl2_ref, out_ref):
    pooled = parts_ref[0]
    for i in range(1, parts_ref.shape[0]):
        pooled = jnp.maximum(pooled, parts_ref[i])
    hid = jnp.maximum(
        jnp.dot(pooled, wl1_ref[...], preferred_element_type=jnp.float32)
        + bl1_ref[...], 0.0)
    out_ref[...] = jnp.sum(hid * wl2_ref[...], axis=1, keepdims=True) + bl2_ref[0, 0]


def _mlp(parts, wl1T, bl1, wl2, bl2):
    g = parts.shape[1]
    return pl.pallas_call(
        _mlp_body,
        out_shape=jax.ShapeDtypeStruct((g, 1), jnp.float32),
    )(parts, wl1T, bl1.reshape(1, -1), wl2, bl2.reshape(1, 1))


def kernel(x, edge_index, batch, edge_weight, Wrel0, brel0, Wroot0, Wrel1, brel1, Wroot1, Wrel2, brel2, Wroot2, Wl1, bl1, Wl2, bl2):
    n, fin = x.shape
    e = edge_weight.shape[0]
    hh = Wrel0.shape[0]
    num_graphs = 64
    ept = e // _NW
    nch = ept // _CS

    ngr = nch // _GB
    srcm = edge_index[0].reshape(_NW, ngr, _GB, _CS)
    dstm = edge_index[1].reshape(_NW, ngr, _GB, _CS)
    wm = edge_weight.reshape(_NW, ngr, _GB, _CS)

    rows_blk = 2000

    xr, xo = _mm2(x, Wrel0.T, Wroot0.T, rows_blk)
    p = _sc_scatter(xr, srcm, dstm, wm)
    xr, xo = _combine_mm2(p, xo, brel0.reshape(1, -1), Wrel1.T, Wroot1.T, rows_blk)
    p = _sc_scatter(xr, srcm, dstm, wm)
    xr, xo = _combine_mm2(p, xo, brel1.reshape(1, -1), Wrel2.T, Wroot2.T, rows_blk)
    p = _sc_scatter(xr, srcm, dstm, wm)
    parts = _sc_pool(p, xo, brel2, batch, num_graphs)
    return _mlp(parts, Wl1.T, bl1, Wl2, bl2)


# speedup vs baseline: 14.9374x; 1.0056x over previous
import functools

import jax
import jax.numpy as jnp
from jax import lax
from jax.experimental import pallas as pl
from jax.experimental.pallas import tpu as pltpu
from jax.experimental.pallas import tpu_sc as plsc

_NC = 2
_NS = 16
_NW = _NC * _NS
_CS = 80
_GB = 5
_NIDLE = 7


def _sc_scatter_body(xr_hbm, srcm_hbm, dstm_hbm, wm_hbm, out_hbm,
                     srcr, dstr, dsts, wr, rows_v, acc, gsem, ssem, isem):
    n_pad, h = acc.shape
    rows_per_tile = n_pad // _NS
    ngr = srcm_hbm.shape[1]
    cid = lax.axis_index("c")
    sid = lax.axis_index("s")
    wid = sid * _NC + cid

    def idx_issue(g, s):
        pltpu.async_copy(srcm_hbm.at[wid, g], srcr.at[s], isem.at[s])
        pltpu.async_copy(dstm_hbm.at[wid, g], dstr.at[s], isem.at[s])
        pltpu.async_copy(wm_hbm.at[wid, g], wr.at[s], isem.at[s])

    def idx_drain(s):
        pltpu.make_async_copy(srcm_hbm.at[wid, 0], srcr.at[s], isem.at[s]).wait()
        pltpu.make_async_copy(dstm_hbm.at[wid, 0], dstr.at[s], isem.at[s]).wait()
        pltpu.make_async_copy(wm_hbm.at[wid, 0], wr.at[s], isem.at[s]).wait()

    def gat_issue(r):
        for b in range(_GB):
            pltpu.async_copy(xr_hbm.at[srcr.at[r, b]], rows_v.at[r, b],
                             gsem.at[r])

    def gat_drain(r):
        for b in range(_GB):
            pltpu.make_async_copy(xr_hbm.at[srcr.at[r, 0]], rows_v.at[r, 0],
                                  gsem.at[r]).wait()

    def sca_issue(r):
        for b in range(_GB):
            for k in range(_CS // 16):
                sl = pl.ds(k * 16, 16)
                dsts[r, b, sl] = dstr[r, b, sl]
        for b in range(_GB):
            pltpu.async_copy(rows_v.at[r, b], acc.at[dsts.at[r, b]],
                             ssem.at[r], add=True)

    def sca_drain(r):
        for b in range(_GB):
            pltpu.make_async_copy(rows_v.at[r, 0], acc.at[dsts.at[r, 0]],
                                  ssem.at[r]).wait()

    def scale(r):
        def _sb(b, _):
            for k in range(_CS // 16):
                wvec = wr[r, b, pl.ds(k * 16, 16)]
                for l in range(16):
                    e = k * 16 + l
                    wb = jnp.full((16,), wvec[l], jnp.float32)
                    for c in range(h // 16):
                        sl = pl.ds(c * 16, 16)
                        rows_v[r, b, e, sl] = rows_v[r, b, e, sl] * wb
            return 0
        lax.fori_loop(0, _GB, _sb, 0)

    def proc(g, r, drain_sca=True, fetch_gat=True, fetch_idx=True):
        rn = (r + 1) % 3
        rf = (r + 2) % 3
        if drain_sca:
            sca_drain(rn)
        if fetch_gat:
            idx_drain(rn)
            gat_issue(rn)
        if fetch_idx:
            idx_issue(g + 2, rf)
        gat_drain(r)
        scale(r)
        sca_issue(r)

    idx_issue(0, 0)
    idx_drain(0)
    gat_issue(0)
    idx_issue(1, 1)

    def _zero_row(i, _):
        for c in range(h // 16):
            rows_v[2, 0, i, pl.ds(c * 16, 16)] = jnp.zeros((16,), jnp.float32)
        return 0
    lax.fori_loop(0, _CS, _zero_row, 0)
    for t in range(rows_per_tile // _CS):
        pltpu.sync_copy(rows_v.at[2, 0],
                        acc.at[pl.ds(sid * rows_per_tile + t * _CS, _CS)])
    plsc.subcore_barrier()

    proc(0, 0, drain_sca=False)
    proc(1, 1, drain_sca=False)

    def _steady(i, _):
        g = 3 * i + 2
        proc(g, 2)
        proc(g + 1, 0)
        proc(g + 2, 1)
        return 0
    lax.fori_loop(0, (ngr - 4) // 3, _steady, 0)

    proc(ngr - 2, (ngr - 2) % 3, fetch_idx=False)
    proc(ngr - 1, (ngr - 1) % 3, fetch_gat=False, fetch_idx=False)
    sca_drain((ngr - 2) % 3)
    sca_drain((ngr - 1) % 3)

    plsc.subcore_barrier()
    rsl = pl.ds(sid * rows_per_tile, rows_per_tile)
    pltpu.sync_copy(acc.at[rsl], out_hbm.at[cid, rsl])


def _sc_scatter(xr, srcm, dstm, wm):
    n, h = xr.shape
    ngr = srcm.shape[1]
    cs = srcm.shape[3]
    rpt = -(-(-(-n // _NS)) // cs) * cs
    n_pad = rpt * _NS
    mesh = plsc.VectorSubcoreMesh(core_axis_name="c", subcore_axis_name="s")
    f = pl.kernel(
        _sc_scatter_body, mesh=mesh,
        compiler_params=pltpu.CompilerParams(use_tc_tiling_on_sc=False),
        out_type=jax.ShapeDtypeStruct((_NC, n_pad, h), jnp.float32),
        scratch_types=[
            pltpu.VMEM((3, _GB, cs), jnp.int32),
            pltpu.VMEM((3, _GB, cs), jnp.int32),
            pltpu.VMEM((3, _GB, cs), jnp.int32),
            pltpu.VMEM((3, _GB, cs), jnp.float32),
            pltpu.VMEM((3, _GB, cs, h), jnp.float32),
            pltpu.VMEM_SHARED((n_pad, h), jnp.float32),
            pltpu.SemaphoreType.DMA((3,)),
            pltpu.SemaphoreType.DMA((3,)),
            pltpu.SemaphoreType.DMA((3,)),
        ],
    )
    return f(xr, srcm, dstm, wm)


def _dotT(x, w):
    return lax.dot_general(x, w, (((1,), (1,)), ((), ())),
                           preferred_element_type=jnp.float32)


def _mm2_body(x_ref, a_ref, b_ref, xr_ref, xo_ref):
    x = x_ref[...]
    xr_ref[...] = _dotT(x, a_ref[...])
    xo_ref[...] = _dotT(x, b_ref[...])


def _mm2(x, wrel, wroot, rows_blk):
    n, fin = x.shape
    hh = wrel.shape[0]
    grid = n // rows_blk
    return pl.pallas_call(
        _mm2_body,
        grid=(grid,),
        in_specs=[
            pl.BlockSpec((rows_blk, fin), lambda i: (i, 0)),
            pl.BlockSpec((hh, fin), lambda i: (0, 0)),
            pl.BlockSpec((hh, fin), lambda i: (0, 0)),
        ],
        out_specs=[
            pl.BlockSpec((rows_blk, hh), lambda i: (i, 0)),
            pl.BlockSpec((rows_blk, hh), lambda i: (i, 0)),
        ],
        out_shape=[
            jax.ShapeDtypeStruct((n, hh), jnp.float32),
            jax.ShapeDtypeStruct((n, hh), jnp.float32),
        ],
    )(x, wrel, wroot)


def _combine_mm2_body(p_ref, xo_ref, b_ref, a2_ref, b2_ref, xr_ref, xo2_ref):
    hcur = jnp.maximum(p_ref[0] + p_ref[1] + xo_ref[...] + b_ref[...], 0.0)
    xr_ref[...] = _dotT(hcur, a2_ref[...])
    xo2_ref[...] = _dotT(hcur, b2_ref[...])


def _combine_mm2(p, xo, brel, wrel, wroot, rows_blk):
    n, hh = xo.shape
    grid = n // rows_blk
    return pl.pallas_call(
        _combine_mm2_body,
        grid=(grid,),
        in_specs=[
            pl.BlockSpec((_NC, rows_blk, hh), lambda i: (0, i, 0)),
            pl.BlockSpec((rows_blk, hh), lambda i: (i, 0)),
            pl.BlockSpec((1, hh), lambda i: (0, 0)),
            pl.BlockSpec((hh, hh), lambda i: (0, 0)),
            pl.BlockSpec((hh, hh), lambda i: (0, 0)),
        ],
        out_specs=[
            pl.BlockSpec((rows_blk, hh), lambda i: (i, 0)),
            pl.BlockSpec((rows_blk, hh), lambda i: (i, 0)),
        ],
        out_shape=[
            jax.ShapeDtypeStruct((n, hh), jnp.float32),
            jax.ShapeDtypeStruct((n, hh), jnp.float32),
        ],
    )(p, xo, brel, wrel, wroot)


def _sc_pool_body(p_hbm, xo_hbm, brel_hbm, batch_hbm, out_hbm,
                  p0_v, p1_v, xo_v, b_v, bid_v, part_v, sem0, sem1, sem2, sem3):
    rows_pw, h = p0_v.shape
    ng = part_v.shape[0]
    cid = lax.axis_index("c")
    sid = lax.axis_index("s")
    wid = sid * _NC + cid
    nact = _NW - _NIDLE
    ninf = jnp.full((16,), -jnp.inf, jnp.float32)

    for i in range(ng):
        for c in range(h // 16):
            part_v[i, pl.ds(c * 16, 16)] = ninf

    @pl.when(wid < nact)
    def _active():
        base = wid * rows_pw
        rs = pl.ds(base, rows_pw)
        cp = [pltpu.async_copy(p_hbm.at[0, rs], p0_v, sem0),
              pltpu.async_copy(p_hbm.at[1, rs], p1_v, sem1),
              pltpu.async_copy(xo_hbm.at[rs], xo_v, sem2),
              pltpu.async_copy(batch_hbm.at[rs], bid_v, sem3)]
        pltpu.sync_copy(brel_hbm, b_v)
        for hdl in cp:
            hdl.wait()

        ids0 = bid_v[pl.ds(0, 16)]

        def _row16(k, carry):
            ids16 = bid_v[pl.ds(k * 16, 16)]
            for l in range(16):
                prev = carry[0]
                m = carry[1:]
                gid = ids16[l]
                newseg = gid != prev
                nsv = jnp.full((16,), newseg)
                for c in range(h // 16):
                    cols = c * 16 + lax.iota(jnp.int32, 16)
                    rows = jnp.full((16,), prev, jnp.int32)
                    plsc.store_scatter(part_v, [rows, cols], m[c])


                i = k * 16 + l
                mn = []
                for c in range(h // 16):
                    sl = pl.ds(c * 16, 16)
                    hrow = jnp.maximum(
                        p0_v[i, sl] + p1_v[i, sl] + xo_v[i, sl] + b_v[0, sl],
                        0.0)
                    mc = jnp.where(nsv, ninf, m[c])
                    mn.append(jnp.maximum(mc, hrow))
                carry = (gid,) + tuple(mn)
            return carry

        carry = lax.fori_loop(0, rows_pw // 16, _row16,
                              (ids0[0],) + tuple(ninf for _ in range(h // 16)))
        prev = carry[0]
        for c in range(h // 16):
            cols = c * 16 + lax.iota(jnp.int32, 16)
            rows = jnp.full((16,), prev, jnp.int32)
            plsc.store_scatter(part_v, [rows, cols], carry[1 + c])

    pltpu.sync_copy(part_v, out_hbm.at[wid])


def _sc_pool(p, xo, brel, batch, num_graphs):
    n, hh = xo.shape
    nact = _NW - _NIDLE
    rows_pw = n // nact
    mesh = plsc.VectorSubcoreMesh(core_axis_name="c", subcore_axis_name="s")
    f = pl.kernel(
        _sc_pool_body, mesh=mesh,
        compiler_params=pltpu.CompilerParams(use_tc_tiling_on_sc=False,
                                             needs_layout_passes=False),
        out_type=jax.ShapeDtypeStruct((_NW, num_graphs, hh), jnp.float32),
        scratch_types=[
            pltpu.VMEM((rows_pw, hh), jnp.float32),
            pltpu.VMEM((rows_pw, hh), jnp.float32),
            pltpu.VMEM((rows_pw, hh), jnp.float32),
            pltpu.VMEM((1, hh), jnp.float32),
            pltpu.VMEM((rows_pw,), jnp.int32),
            pltpu.VMEM((num_graphs, hh), jnp.float32),
            pltpu.SemaphoreType.DMA,
            pltpu.SemaphoreType.DMA,
            pltpu.SemaphoreType.DMA,
            pltpu.SemaphoreType.DMA,
        ],
    )
    return f(p, xo, brel.reshape(1, -1), batch)


def _mlp_body(parts_ref, wl1_ref, bl1_ref, wl2_ref, bl2_ref, out_ref):
    pooled = parts_ref[0]
    for i in range(1, parts_ref.shape[0]):
        pooled = jnp.maximum(pooled, parts_ref[i])
    hid = jnp.maximum(_dotT(pooled, wl1_ref[...]) + bl1_ref[...], 0.0)
    out_ref[...] = jnp.sum(hid * wl2_ref[...], axis=1, keepdims=True) + bl2_ref[0, 0]


def _mlp(parts, wl1, bl1, wl2, bl2):
    g = parts.shape[1]
    return pl.pallas_call(
        _mlp_body,
        out_shape=jax.ShapeDtypeStruct((g, 1), jnp.float32),
    )(parts, wl1, bl1.reshape(1, -1), wl2, bl2.reshape(1, 1))


def kernel(x, edge_index, batch, edge_weight, Wrel0, brel0, Wroot0, Wrel1, brel1, Wroot1, Wrel2, brel2, Wroot2, Wl1, bl1, Wl2, bl2):
    n, fin = x.shape
    e = edge_weight.shape[0]
    hh = Wrel0.shape[0]
    num_graphs = 64
    ept = e // _NW
    nch = ept // _CS

    ngr = nch // _GB
    srcm = edge_index[0].reshape(_NW, ngr, _GB, _CS)
    dstm = edge_index[1].reshape(_NW, ngr, _GB, _CS)
    wm = edge_weight.reshape(_NW, ngr, _GB, _CS)

    rows_blk = 2000

    xr, xo = _mm2(x, Wrel0, Wroot0, rows_blk)
    p = _sc_scatter(xr, srcm, dstm, wm)
    xr, xo = _combine_mm2(p, xo, brel0.reshape(1, -1), Wrel1, Wroot1, rows_blk)
    p = _sc_scatter(xr, srcm, dstm, wm)
    xr, xo = _combine_mm2(p, xo, brel1.reshape(1, -1), Wrel2, Wroot2, rows_blk)
    p = _sc_scatter(xr, srcm, dstm, wm)
    parts = _sc_pool(p, xo, brel2, batch, num_graphs)
    return _mlp(parts, Wl1, bl1, Wl2, bl2)


# speedup vs baseline: 15.1789x; 1.0162x over previous
import functools

import jax
import jax.numpy as jnp
from jax import lax
from jax.experimental import pallas as pl
from jax.experimental.pallas import tpu as pltpu
from jax.experimental.pallas import tpu_sc as plsc

_NC = 2
_NS = 16
_NW = _NC * _NS
_CS = 80
_GB = 5
_NIDLE = 7


def _sc_scatter_body(xr_hbm, srcm_hbm, dstm_hbm, wm_hbm, out_hbm,
                     srcr, dstr, dsts, wr, rows_v, acc, gsem, ssem, isem):
    n_pad, h = acc.shape
    rows_per_tile = n_pad // _NS
    ngr = srcm_hbm.shape[1]
    cid = lax.axis_index("c")
    sid = lax.axis_index("s")
    wid = sid * _NC + cid

    def idx_issue(g, s):
        pltpu.async_copy(srcm_hbm.at[wid, g], srcr.at[s], isem.at[s])
        pltpu.async_copy(dstm_hbm.at[wid, g], dstr.at[s], isem.at[s])
        pltpu.async_copy(wm_hbm.at[wid, g], wr.at[s], isem.at[s])

    def idx_drain(s):
        pltpu.make_async_copy(srcm_hbm.at[wid, 0], srcr.at[s], isem.at[s]).wait()
        pltpu.make_async_copy(dstm_hbm.at[wid, 0], dstr.at[s], isem.at[s]).wait()
        pltpu.make_async_copy(wm_hbm.at[wid, 0], wr.at[s], isem.at[s]).wait()

    def gat_issue(r):
        for b in range(_GB):
            pltpu.async_copy(xr_hbm.at[srcr.at[r, b]], rows_v.at[r, b],
                             gsem.at[r])

    def gat_drain(r):
        for b in range(_GB):
            pltpu.make_async_copy(xr_hbm.at[srcr.at[r, 0]], rows_v.at[r, 0],
                                  gsem.at[r]).wait()

    def sca_drain(r):
        for b in range(_GB):
            pltpu.make_async_copy(rows_v.at[r, 0], acc.at[dsts.at[r, 0]],
                                  ssem.at[r]).wait()

    def scale_scatter(r):
        for b in range(_GB):
            for k in range(_CS // 16):
                sl = pl.ds(k * 16, 16)
                dsts[r, b, sl] = dstr[r, b, sl]

        def _sb(b, _):
            for k in range(_CS // 16):
                wvec = wr[r, b, pl.ds(k * 16, 16)]
                for l in range(16):
                    e = k * 16 + l
                    wb = jnp.full((16,), wvec[l], jnp.float32)
                    for c in range(h // 16):
                        sl = pl.ds(c * 16, 16)
                        rows_v[r, b, e, sl] = rows_v[r, b, e, sl] * wb
            pltpu.async_copy(rows_v.at[r, b], acc.at[dsts.at[r, b]],
                             ssem.at[r], add=True)
            return 0
        lax.fori_loop(0, _GB, _sb, 0)

    def proc(g, r, drain_sca=True, fetch_gat=True, fetch_idx=True):
        rn = (r + 1) % 3
        rf = (r + 2) % 3
        if drain_sca:
            sca_drain(rn)
        if fetch_gat:
            idx_drain(rn)
            gat_issue(rn)
        if fetch_idx:
            idx_issue(g + 2, rf)
        gat_drain(r)
        scale_scatter(r)

    idx_issue(0, 0)
    idx_drain(0)
    gat_issue(0)
    idx_issue(1, 1)

    def _zero_row(i, _):
        for c in range(h // 16):
            rows_v[2, 0, i, pl.ds(c * 16, 16)] = jnp.zeros((16,), jnp.float32)
        return 0
    lax.fori_loop(0, _CS, _zero_row, 0)
    for t in range(rows_per_tile // _CS):
        pltpu.async_copy(rows_v.at[2, 0],
                         acc.at[pl.ds(sid * rows_per_tile + t * _CS, _CS)],
                         ssem.at[0])
    for t in range(rows_per_tile // _CS):
        pltpu.make_async_copy(rows_v.at[2, 0],
                              acc.at[pl.ds(sid * rows_per_tile, _CS)],
                              ssem.at[0]).wait()
    plsc.subcore_barrier()

    proc(0, 0, drain_sca=False)
    proc(1, 1, drain_sca=False)

    def _steady(i, _):
        g = 3 * i + 2
        proc(g, 2)
        proc(g + 1, 0)
        proc(g + 2, 1)
        return 0
    lax.fori_loop(0, (ngr - 4) // 3, _steady, 0)

    proc(ngr - 2, (ngr - 2) % 3, fetch_idx=False)
    proc(ngr - 1, (ngr - 1) % 3, fetch_gat=False, fetch_idx=False)
    sca_drain((ngr - 2) % 3)
    sca_drain((ngr - 1) % 3)

    plsc.subcore_barrier()
    rsl = pl.ds(sid * rows_per_tile, rows_per_tile)
    pltpu.sync_copy(acc.at[rsl], out_hbm.at[cid, rsl])


def _sc_scatter(xr, srcm, dstm, wm):
    n, h = xr.shape
    ngr = srcm.shape[1]
    cs = srcm.shape[3]
    rpt = -(-(-(-n // _NS)) // cs) * cs
    n_pad = rpt * _NS
    mesh = plsc.VectorSubcoreMesh(core_axis_name="c", subcore_axis_name="s")
    f = pl.kernel(
        _sc_scatter_body, mesh=mesh,
        compiler_params=pltpu.CompilerParams(use_tc_tiling_on_sc=False),
        out_type=jax.ShapeDtypeStruct((_NC, n_pad, h), jnp.float32),
        scratch_types=[
            pltpu.VMEM((3, _GB, cs), jnp.int32),
            pltpu.VMEM((3, _GB, cs), jnp.int32),
            pltpu.VMEM((3, _GB, cs), jnp.int32),
            pltpu.VMEM((3, _GB, cs), jnp.float32),
            pltpu.VMEM((3, _GB, cs, h), jnp.float32),
            pltpu.VMEM_SHARED((n_pad, h), jnp.float32),
            pltpu.SemaphoreType.DMA((3,)),
            pltpu.SemaphoreType.DMA((3,)),
            pltpu.SemaphoreType.DMA((3,)),
        ],
    )
    return f(xr, srcm, dstm, wm)


def _dotT(x, w):
    return lax.dot_general(x, w, (((1,), (1,)), ((), ())),
                           preferred_element_type=jnp.float32)


def _mm2_body(x_ref, a_ref, b_ref, xr_ref, xo_ref):
    x = x_ref[...]
    xr_ref[...] = _dotT(x, a_ref[...])
    xo_ref[...] = _dotT(x, b_ref[...])


def _mm2(x, wrel, wroot, rows_blk):
    n, fin = x.shape
    hh = wrel.shape[0]
    grid = n // rows_blk
    return pl.pallas_call(
        _mm2_body,
        grid=(grid,),
        in_specs=[
            pl.BlockSpec((rows_blk, fin), lambda i: (i, 0)),
            pl.BlockSpec((hh, fin), lambda i: (0, 0)),
            pl.BlockSpec((hh, fin), lambda i: (0, 0)),
        ],
        out_specs=[
            pl.BlockSpec((rows_blk, hh), lambda i: (i, 0)),
            pl.BlockSpec((rows_blk, hh), lambda i: (i, 0)),
        ],
        out_shape=[
            jax.ShapeDtypeStruct((n, hh), jnp.float32),
            jax.ShapeDtypeStruct((n, hh), jnp.float32),
        ],
    )(x, wrel, wroot)


def _combine_mm2_body(p_ref, xo_ref, b_ref, a2_ref, b2_ref, xr_ref, xo2_ref):
    hcur = jnp.maximum(p_ref[0] + p_ref[1] + xo_ref[...] + b_ref[...], 0.0)
    xr_ref[...] = _dotT(hcur, a2_ref[...])
    xo2_ref[...] = _dotT(hcur, b2_ref[...])


def _combine_mm2(p, xo, brel, wrel, wroot, rows_blk):
    n, hh = xo.shape
    grid = n // rows_blk
    return pl.pallas_call(
        _combine_mm2_body,
        grid=(grid,),
        in_specs=[
            pl.BlockSpec((_NC, rows_blk, hh), lambda i: (0, i, 0)),
            pl.BlockSpec((rows_blk, hh), lambda i: (i, 0)),
            pl.BlockSpec((1, hh), lambda i: (0, 0)),
            pl.BlockSpec((hh, hh), lambda i: (0, 0)),
            pl.BlockSpec((hh, hh), lambda i: (0, 0)),
        ],
        out_specs=[
            pl.BlockSpec((rows_blk, hh), lambda i: (i, 0)),
            pl.BlockSpec((rows_blk, hh), lambda i: (i, 0)),
        ],
        out_shape=[
            jax.ShapeDtypeStruct((n, hh), jnp.float32),
            jax.ShapeDtypeStruct((n, hh), jnp.float32),
        ],
    )(p, xo, brel, wrel, wroot)


def _sc_pool_body(p_hbm, xo_hbm, brel_hbm, batch_hbm, out_hbm,
                  p0_v, p1_v, xo_v, b_v, bid_v, part_v, sem0, sem1, sem2, sem3):
    rows_pw, h = p0_v.shape
    ng = part_v.shape[0]
    cid = lax.axis_index("c")
    sid = lax.axis_index("s")
    wid = sid * _NC + cid
    nact = _NW - _NIDLE
    ninf = jnp.full((16,), -jnp.inf, jnp.float32)

    for i in range(ng):
        for c in range(h // 16):
            part_v[i, pl.ds(c * 16, 16)] = ninf

    @pl.when(wid < nact)
    def _active():
        base = wid * rows_pw
        rs = pl.ds(base, rows_pw)
        cp = [pltpu.async_copy(p_hbm.at[0, rs], p0_v, sem0),
              pltpu.async_copy(p_hbm.at[1, rs], p1_v, sem1),
              pltpu.async_copy(xo_hbm.at[rs], xo_v, sem2),
              pltpu.async_copy(batch_hbm.at[rs], bid_v, sem3)]
        pltpu.sync_copy(brel_hbm, b_v)
        for hdl in cp:
            hdl.wait()

        ids0 = bid_v[pl.ds(0, 16)]

        def _row16(k, carry):
            ids16 = bid_v[pl.ds(k * 16, 16)]
            for l in range(16):
                prev = carry[0]
                m = carry[1:]
                gid = ids16[l]
                newseg = gid != prev
                nsv = jnp.full((16,), newseg)
                for c in range(h // 16):
                    cols = c * 16 + lax.iota(jnp.int32, 16)
                    rows = jnp.full((16,), prev, jnp.int32)
                    plsc.store_scatter(part_v, [rows, cols], m[c])


                i = k * 16 + l
                mn = []
                for c in range(h // 16):
                    sl = pl.ds(c * 16, 16)
                    hrow = jnp.maximum(
                        p0_v[i, sl] + p1_v[i, sl] + xo_v[i, sl] + b_v[0, sl],
                        0.0)
                    mc = jnp.where(nsv, ninf, m[c])
                    mn.append(jnp.maximum(mc, hrow))
                carry = (gid,) + tuple(mn)
            return carry

        carry = lax.fori_loop(0, rows_pw // 16, _row16,
                              (ids0[0],) + tuple(ninf for _ in range(h // 16)))
        prev = carry[0]
        for c in range(h // 16):
            cols = c * 16 + lax.iota(jnp.int32, 16)
            rows = jnp.full((16,), prev, jnp.int32)
            plsc.store_scatter(part_v, [rows, cols], carry[1 + c])

    pltpu.sync_copy(part_v, out_hbm.at[wid])


def _sc_pool(p, xo, brel, batch, num_graphs):
    n, hh = xo.shape
    nact = _NW - _NIDLE
    rows_pw = n // nact
    mesh = plsc.VectorSubcoreMesh(core_axis_name="c", subcore_axis_name="s")
    f = pl.kernel(
        _sc_pool_body, mesh=mesh,
        compiler_params=pltpu.CompilerParams(use_tc_tiling_on_sc=False,
                                             needs_layout_passes=False),
        out_type=jax.ShapeDtypeStruct((_NW, num_graphs, hh), jnp.float32),
        scratch_types=[
            pltpu.VMEM((rows_pw, hh), jnp.float32),
            pltpu.VMEM((rows_pw, hh), jnp.float32),
            pltpu.VMEM((rows_pw, hh), jnp.float32),
            pltpu.VMEM((1, hh), jnp.float32),
            pltpu.VMEM((rows_pw,), jnp.int32),
            pltpu.VMEM((num_graphs, hh), jnp.float32),
            pltpu.SemaphoreType.DMA,
            pltpu.SemaphoreType.DMA,
            pltpu.SemaphoreType.DMA,
            pltpu.SemaphoreType.DMA,
        ],
    )
    return f(p, xo, brel.reshape(1, -1), batch)


def _mlp_body(parts_ref, wl1_ref, bl1_ref, wl2_ref, bl2_ref, out_ref):
    pooled = parts_ref[0]
    for i in range(1, parts_ref.shape[0]):
        pooled = jnp.maximum(pooled, parts_ref[i])
    hid = jnp.maximum(_dotT(pooled, wl1_ref[...]) + bl1_ref[...], 0.0)
    out_ref[...] = jnp.sum(hid * wl2_ref[...], axis=1, keepdims=True) + bl2_ref[0, 0]


def _mlp(parts, wl1, bl1, wl2, bl2):
    g = parts.shape[1]
    return pl.pallas_call(
        _mlp_body,
        out_shape=jax.ShapeDtypeStruct((g, 1), jnp.float32),
    )(parts, wl1, bl1.reshape(1, -1), wl2, bl2.reshape(1, 1))


def kernel(x, edge_index, batch, edge_weight, Wrel0, brel0, Wroot0, Wrel1, brel1, Wroot1, Wrel2, brel2, Wroot2, Wl1, bl1, Wl2, bl2):
    n, fin = x.shape
    e = edge_weight.shape[0]
    hh = Wrel0.shape[0]
    num_graphs = 64
    ept = e // _NW
    nch = ept // _CS

    ngr = nch // _GB
    srcm = edge_index[0].reshape(_NW, ngr, _GB, _CS)
    dstm = edge_index[1].reshape(_NW, ngr, _GB, _CS)
    wm = edge_weight.reshape(_NW, ngr, _GB, _CS)

    rows_blk = 2000

    xr, xo = _mm2(x, Wrel0, Wroot0, rows_blk)
    p = _sc_scatter(xr, srcm, dstm, wm)
    xr, xo = _combine_mm2(p, xo, brel0.reshape(1, -1), Wrel1, Wroot1, rows_blk)
    p = _sc_scatter(xr, srcm, dstm, wm)
    xr, xo = _combine_mm2(p, xo, brel1.reshape(1, -1), Wrel2, Wroot2, rows_blk)
    p = _sc_scatter(xr, srcm, dstm, wm)
    parts = _sc_pool(p, xo, brel2, batch, num_graphs)
    return _mlp(parts, Wl1, bl1, Wl2, bl2)


# speedup vs baseline: 15.2010x; 1.0015x over previous
import jax
import jax.numpy as jnp
from jax import lax
from jax.experimental import pallas as pl
from jax.experimental.pallas import tpu as pltpu
from jax.experimental.pallas import tpu_sc as plsc

_NC = 2
_NS = 16
_NW = _NC * _NS
_CS = 80
_GB = 5
_NIDLE = 7


def _sc_scatter_body(xr_hbm, srcm_hbm, dstm_hbm, wm_hbm, out_hbm,
                     srcr, dstr, dsts, wr, rows_v, acc, gsem, ssem, isem):
    n_pad, h = acc.shape
    rows_per_tile = n_pad // _NS
    ngr = srcm_hbm.shape[1]
    cid = lax.axis_index("c")
    sid = lax.axis_index("s")
    wid = sid * _NC + cid

    def idx_issue(g, s):
        pltpu.async_copy(srcm_hbm.at[wid, g], srcr.at[s], isem.at[s])
        pltpu.async_copy(dstm_hbm.at[wid, g], dstr.at[s], isem.at[s])
        pltpu.async_copy(wm_hbm.at[wid, g], wr.at[s], isem.at[s])

    def idx_drain(s):
        pltpu.make_async_copy(srcm_hbm.at[wid, 0], srcr.at[s], isem.at[s]).wait()
        pltpu.make_async_copy(dstm_hbm.at[wid, 0], dstr.at[s], isem.at[s]).wait()
        pltpu.make_async_copy(wm_hbm.at[wid, 0], wr.at[s], isem.at[s]).wait()

    def gat_issue(r):
        for b in range(_GB):
            pltpu.async_copy(xr_hbm.at[srcr.at[r, b]], rows_v.at[r, b],
                             gsem.at[r])

    def gat_drain(r):
        for b in range(_GB):
            pltpu.make_async_copy(xr_hbm.at[srcr.at[r, 0]], rows_v.at[r, 0],
                                  gsem.at[r]).wait()

    def sca_drain(r):
        for b in range(_GB):
            pltpu.make_async_copy(rows_v.at[r, 0], acc.at[dsts.at[r, 0]],
                                  ssem.at[r]).wait()

    def scale_scatter(r):
        for b in range(_GB):
            for k in range(_CS // 16):
                sl = pl.ds(k * 16, 16)
                dsts[r, b, sl] = dstr[r, b, sl]

        def _sb(b, _):
            for k in range(_CS // 16):
                wvec = wr[r, b, pl.ds(k * 16, 16)]
                for l in range(16):
                    e = k * 16 + l
                    wb = jnp.full((16,), wvec[l], jnp.float32)
                    for c in range(h // 16):
                        sl = pl.ds(c * 16, 16)
                        rows_v[r, b, e, sl] = rows_v[r, b, e, sl] * wb
            pltpu.async_copy(rows_v.at[r, b], acc.at[dsts.at[r, b]],
                             ssem.at[r], add=True)
            return 0
        lax.fori_loop(0, _GB, _sb, 0)

    def proc(g, r, drain_sca=True, fetch_gat=True, fetch_idx=True):
        rn = (r + 1) % 3
        rf = (r + 2) % 3
        if drain_sca:
            sca_drain(rn)
        if fetch_gat:
            idx_drain(rn)
            gat_issue(rn)
        if fetch_idx:
            idx_issue(g + 2, rf)
        gat_drain(r)
        scale_scatter(r)

    idx_issue(0, 0)
    idx_drain(0)
    gat_issue(0)
    idx_issue(1, 1)

    def _zero_row(i, _):
        for c in range(h // 16):
            rows_v[2, 0, i, pl.ds(c * 16, 16)] = jnp.zeros((16,), jnp.float32)
        return 0
    lax.fori_loop(0, _CS, _zero_row, 0)
    for t in range(rows_per_tile // _CS):
        pltpu.async_copy(rows_v.at[2, 0],
                         acc.at[pl.ds(sid * rows_per_tile + t * _CS, _CS)],
                         ssem.at[0])
    for t in range(rows_per_tile // _CS):
        pltpu.make_async_copy(rows_v.at[2, 0],
                              acc.at[pl.ds(sid * rows_per_tile, _CS)],
                              ssem.at[0]).wait()
    plsc.subcore_barrier()

    proc(0, 0, drain_sca=False)
    proc(1, 1, drain_sca=False)

    def _steady(i, _):
        g = 3 * i + 2
        proc(g, 2)
        proc(g + 1, 0)
        proc(g + 2, 1)
        return 0
    lax.fori_loop(0, (ngr - 4) // 3, _steady, 0)

    proc(ngr - 2, (ngr - 2) % 3, fetch_idx=False)
    proc(ngr - 1, (ngr - 1) % 3, fetch_gat=False, fetch_idx=False)
    sca_drain((ngr - 2) % 3)
    sca_drain((ngr - 1) % 3)

    plsc.subcore_barrier()
    rsl = pl.ds(sid * rows_per_tile, rows_per_tile)
    pltpu.sync_copy(acc.at[rsl], out_hbm.at[cid, rsl])


def _sc_scatter(xr, srcm, dstm, wm):
    n, h = xr.shape
    ngr = srcm.shape[1]
    cs = srcm.shape[3]
    rpt = -(-(-(-n // _NS)) // cs) * cs
    n_pad = rpt * _NS
    mesh = plsc.VectorSubcoreMesh(core_axis_name="c", subcore_axis_name="s")
    f = pl.kernel(
        _sc_scatter_body, mesh=mesh,
        compiler_params=pltpu.CompilerParams(use_tc_tiling_on_sc=False),
        out_type=jax.ShapeDtypeStruct((_NC, n_pad, h), jnp.float32),
        scratch_types=[
            pltpu.VMEM((3, _GB, cs), jnp.int32),
            pltpu.VMEM((3, _GB, cs), jnp.int32),
            pltpu.VMEM((3, _GB, cs), jnp.int32),
            pltpu.VMEM((3, _GB, cs), jnp.float32),
            pltpu.VMEM((3, _GB, cs, h), jnp.float32),
            pltpu.VMEM_SHARED((n_pad, h), jnp.float32),
            pltpu.SemaphoreType.DMA((3,)),
            pltpu.SemaphoreType.DMA((3,)),
            pltpu.SemaphoreType.DMA((3,)),
        ],
    )
    return f(xr, srcm, dstm, wm)


def _dotT(x, w):
    return lax.dot_general(x, w, (((1,), (1,)), ((), ())),
                           preferred_element_type=jnp.float32)


def _mm2_body(x_ref, a_ref, b_ref, xr_ref, xo_ref):
    x = x_ref[...]
    xr_ref[...] = _dotT(x, a_ref[...])
    xo_ref[...] = _dotT(x, b_ref[...])


def _mm2(x, wrel, wroot, rows_blk):
    n, fin = x.shape
    hh = wrel.shape[0]
    grid = n // rows_blk
    return pl.pallas_call(
        _mm2_body,
        grid=(grid,),
        in_specs=[
            pl.BlockSpec((rows_blk, fin), lambda i: (i, 0)),
            pl.BlockSpec((hh, fin), lambda i: (0, 0)),
            pl.BlockSpec((hh, fin), lambda i: (0, 0)),
        ],
        out_specs=[
            pl.BlockSpec((rows_blk, hh), lambda i: (i, 0)),
            pl.BlockSpec((rows_blk, hh), lambda i: (i, 0)),
        ],
        out_shape=[
            jax.ShapeDtypeStruct((n, hh), jnp.float32),
            jax.ShapeDtypeStruct((n, hh), jnp.float32),
        ],
    )(x, wrel, wroot)


def _combine_mm2_body(p_ref, xo_ref, b_ref, a2_ref, b2_ref, xr_ref, xo2_ref):
    hcur = jnp.maximum(p_ref[0] + p_ref[1] + xo_ref[...] + b_ref[...], 0.0)
    xr_ref[...] = _dotT(hcur, a2_ref[...])
    xo2_ref[...] = _dotT(hcur, b2_ref[...])


def _combine_mm2(p, xo, brel, wrel, wroot, rows_blk):
    n, hh = xo.shape
    grid = n // rows_blk
    return pl.pallas_call(
        _combine_mm2_body,
        grid=(grid,),
        in_specs=[
            pl.BlockSpec((_NC, rows_blk, hh), lambda i: (0, i, 0)),
            pl.BlockSpec((rows_blk, hh), lambda i: (i, 0)),
            pl.BlockSpec((1, hh), lambda i: (0, 0)),
            pl.BlockSpec((hh, hh), lambda i: (0, 0)),
            pl.BlockSpec((hh, hh), lambda i: (0, 0)),
        ],
        out_specs=[
            pl.BlockSpec((rows_blk, hh), lambda i: (i, 0)),
            pl.BlockSpec((rows_blk, hh), lambda i: (i, 0)),
        ],
        out_shape=[
            jax.ShapeDtypeStruct((n, hh), jnp.float32),
            jax.ShapeDtypeStruct((n, hh), jnp.float32),
        ],
    )(p, xo, brel, wrel, wroot)


def _sc_pool_body(p_hbm, xo_hbm, brel_hbm, batch_hbm, out_hbm,
                  p0_v, p1_v, xo_v, b_v, bid_v, part_v, sem0, sem1, sem2, sem3):
    rows_pw, h = p0_v.shape
    ng = part_v.shape[0]
    cid = lax.axis_index("c")
    sid = lax.axis_index("s")
    wid = sid * _NC + cid
    nact = _NW - _NIDLE
    ninf = jnp.full((16,), -jnp.inf, jnp.float32)

    for i in range(ng):
        for c in range(h // 16):
            part_v[i, pl.ds(c * 16, 16)] = ninf

    @pl.when(wid < nact)
    def _active():
        base = wid * rows_pw
        rs = pl.ds(base, rows_pw)
        cp = [pltpu.async_copy(p_hbm.at[0, rs], p0_v, sem0),
              pltpu.async_copy(p_hbm.at[1, rs], p1_v, sem1),
              pltpu.async_copy(xo_hbm.at[rs], xo_v, sem2),
              pltpu.async_copy(batch_hbm.at[rs], bid_v, sem3)]
        pltpu.sync_copy(brel_hbm, b_v)
        for hdl in cp:
            hdl.wait()

        ids0 = bid_v[pl.ds(0, 16)]

        def _row16(k, carry):
            ids16 = bid_v[pl.ds(k * 16, 16)]
            for l in range(16):
                prev = carry[0]
                m = carry[1:]
                gid = ids16[l]
                newseg = gid != prev
                nsv = jnp.full((16,), newseg)
                for c in range(h // 16):
                    cols = c * 16 + lax.iota(jnp.int32, 16)
                    rows = jnp.full((16,), prev, jnp.int32)
                    plsc.store_scatter(part_v, [rows, cols], m[c])


                i = k * 16 + l
                mn = []
                for c in range(h // 16):
                    sl = pl.ds(c * 16, 16)
                    hrow = jnp.maximum(
                        p0_v[i, sl] + p1_v[i, sl] + xo_v[i, sl] + b_v[0, sl],
                        0.0)
                    mc = jnp.where(nsv, ninf, m[c])
                    mn.append(jnp.maximum(mc, hrow))
                carry = (gid,) + tuple(mn)
            return carry

        carry = lax.fori_loop(0, rows_pw // 16, _row16,
                              (ids0[0],) + tuple(ninf for _ in range(h // 16)))
        prev = carry[0]
        for c in range(h // 16):
            cols = c * 16 + lax.iota(jnp.int32, 16)
            rows = jnp.full((16,), prev, jnp.int32)
            plsc.store_scatter(part_v, [rows, cols], carry[1 + c])

    pltpu.sync_copy(part_v, out_hbm.at[wid])


def _sc_pool(p, xo, brel, batch, num_graphs):
    n, hh = xo.shape
    nact = _NW - _NIDLE
    rows_pw = n // nact
    mesh = plsc.VectorSubcoreMesh(core_axis_name="c", subcore_axis_name="s")
    f = pl.kernel(
        _sc_pool_body, mesh=mesh,
        compiler_params=pltpu.CompilerParams(use_tc_tiling_on_sc=False,
                                             needs_layout_passes=False),
        out_type=jax.ShapeDtypeStruct((_NW, num_graphs, hh), jnp.float32),
        scratch_types=[
            pltpu.VMEM((rows_pw, hh), jnp.float32),
            pltpu.VMEM((rows_pw, hh), jnp.float32),
            pltpu.VMEM((rows_pw, hh), jnp.float32),
            pltpu.VMEM((1, hh), jnp.float32),
            pltpu.VMEM((rows_pw,), jnp.int32),
            pltpu.VMEM((num_graphs, hh), jnp.float32),
            pltpu.SemaphoreType.DMA,
            pltpu.SemaphoreType.DMA,
            pltpu.SemaphoreType.DMA,
            pltpu.SemaphoreType.DMA,
        ],
    )
    return f(p, xo, brel.reshape(1, -1), batch)


def _mlp_body(parts_ref, wl1_ref, bl1_ref, wl2_ref, bl2_ref, out_ref):
    pooled = parts_ref[0]
    for i in range(1, parts_ref.shape[0]):
        pooled = jnp.maximum(pooled, parts_ref[i])
    hid = jnp.maximum(_dotT(pooled, wl1_ref[...]) + bl1_ref[...], 0.0)
    out_ref[...] = jnp.sum(hid * wl2_ref[...], axis=1, keepdims=True) + bl2_ref[0, 0]


def _mlp(parts, wl1, bl1, wl2, bl2):
    g = parts.shape[1]
    return pl.pallas_call(
        _mlp_body,
        out_shape=jax.ShapeDtypeStruct((g, 1), jnp.float32),
    )(parts, wl1, bl1.reshape(1, -1), wl2, bl2.reshape(1, 1))


def kernel(x, edge_index, batch, edge_weight, Wrel0, brel0, Wroot0, Wrel1, brel1, Wroot1, Wrel2, brel2, Wroot2, Wl1, bl1, Wl2, bl2):
    n, fin = x.shape
    e = edge_weight.shape[0]
    hh = Wrel0.shape[0]
    num_graphs = 64
    ept = e // _NW
    nch = ept // _CS

    ngr = nch // _GB
    srcm = edge_index[0].reshape(_NW, ngr, _GB, _CS)
    dstm = edge_index[1].reshape(_NW, ngr, _GB, _CS)
    wm = edge_weight.reshape(_NW, ngr, _GB, _CS)

    rows_blk = 2000

    xr, xo = _mm2(x, Wrel0, Wroot0, rows_blk)
    p = _sc_scatter(xr, srcm, dstm, wm)
    xr, xo = _combine_mm2(p, xo, brel0.reshape(1, -1), Wrel1, Wroot1, rows_blk)
    p = _sc_scatter(xr, srcm, dstm, wm)
    xr, xo = _combine_mm2(p, xo, brel1.reshape(1, -1), Wrel2, Wroot2, rows_blk)
    p = _sc_scatter(xr, srcm, dstm, wm)
    parts = _sc_pool(p, xo, brel2, batch, num_graphs)
    return _mlp(parts, Wl1, bl1, Wl2, bl2)
